# Initial kernel scaffold; baseline (speedup 1.0000x reference)
import jax
import jax.numpy as jnp
from jax.experimental import pallas as pl


def kernel(x_embeddings, edge_embeddings, edge_index, W, b):
    raise NotImplementedError("write your pallas kernel here")


# speedup vs baseline: 4.6617x; 4.6617x over previous
import functools

import jax
import jax.numpy as jnp
from jax import lax
from jax.experimental import pallas as pl
from jax.experimental.pallas import tpu as pltpu
from jax.experimental.pallas import tpu_sc as plsc

N_WORKERS = 32
LANES = 16


def _node_scores_body(x_ref, w_ref, o_ref):
    o_ref[...] = jnp.dot(x_ref[...], w_ref[...],
                         preferred_element_type=jnp.float32)


def _edge_scores_body(e_ref, w_ref, b_ref, o_ref):
    o_ref[...] = jnp.dot(e_ref[...], w_ref[...],
                         preferred_element_type=jnp.float32) + b_ref[...]


def _make_sc_gather(e_pad, n_tab):
    chunk = e_pad // N_WORKERS
    iters = chunk // LANES
    mesh = plsc.VectorSubcoreMesh(core_axis_name="c", subcore_axis_name="s")

    @functools.partial(
        pl.kernel,
        mesh=mesh,
        out_type=jax.ShapeDtypeStruct((e_pad,), jnp.float32),
        scratch_types=[
            pltpu.VMEM((n_tab,), jnp.float32),
            pltpu.VMEM((chunk,), jnp.int32),
            pltpu.VMEM((chunk,), jnp.int32),
            pltpu.VMEM((chunk,), jnp.float32),
            pltpu.VMEM((chunk,), jnp.float32),
        ],
        compiler_params=pltpu.CompilerParams(needs_layout_passes=False),
    )
    def sc_gather(tab_hbm, row_hbm, col_hbm, es_hbm, out_hbm,
                  tab_v, row_v, col_v, es_v, out_v):
        wid = lax.axis_index("s") * 2 + lax.axis_index("c")
        base = wid * chunk
        pltpu.sync_copy(tab_hbm, tab_v)
        pltpu.sync_copy(row_hbm.at[pl.ds(base, chunk)], row_v)
        pltpu.sync_copy(col_hbm.at[pl.ds(base, chunk)], col_v)
        pltpu.sync_copy(es_hbm.at[pl.ds(base, chunk)], es_v)

        def body(i, carry):
            off = i * LANES
            ir = row_v[pl.ds(off, LANES)]
            ic = col_v[pl.ds(off, LANES)]
            g_r = plsc.load_gather(tab_v, [ir * 2])
            g_c = plsc.load_gather(tab_v, [ic * 2 + 1])
            out_v[pl.ds(off, LANES)] = g_r + g_c + es_v[pl.ds(off, LANES)]
            return carry

        lax.fori_loop(0, iters, body, 0)
        pltpu.sync_copy(out_v, out_hbm.at[pl.ds(base, chunk)])

    return sc_gather


def kernel(x_embeddings, edge_embeddings, edge_index, W, b):
    n, h = x_embeddings.shape
    e = edge_embeddings.shape[0]

    row = edge_index[0].astype(jnp.int32)
    col = edge_index[1].astype(jnp.int32)
    w12 = jnp.concatenate([W[:h], W[h:2 * h]], axis=1)
    w3 = W[2 * h:]
    b2 = b.reshape(1, 1)

    n_blk = 2000
    ns2d = pl.pallas_call(
        _node_scores_body,
        grid=(n // n_blk,),
        in_specs=[
            pl.BlockSpec((n_blk, h), lambda i: (i, 0)),
            pl.BlockSpec((h, 2), lambda i: (0, 0)),
        ],
        out_specs=pl.BlockSpec((n_blk, 2), lambda i: (i, 0)),
        out_shape=jax.ShapeDtypeStruct((n, 2), jnp.float32),
    )(x_embeddings, w12)
    tab = ns2d.reshape(n * 2)

    e_blk = 1280
    e_pad = -(-e // 2560) * 2560
    es2d = pl.pallas_call(
        _edge_scores_body,
        grid=(e // e_blk,),
        in_specs=[
            pl.BlockSpec((e_blk, h), lambda i: (i, 0)),
            pl.BlockSpec((h, 1), lambda i: (0, 0)),
            pl.BlockSpec((1, 1), lambda i: (0, 0)),
        ],
        out_specs=pl.BlockSpec((e_blk, 1), lambda i: (i, 0)),
        out_shape=jax.ShapeDtypeStruct((e_pad, 1), jnp.float32),
    )(edge_embeddings, w3, b2)
    es = es2d.reshape(e_pad)

    pad = e_pad - e
    zpad = jnp.zeros((pad,), jnp.int32)
    row_p = jnp.concatenate([row, zpad])
    col_p = jnp.concatenate([col, zpad])

    out_p = _make_sc_gather(e_pad, n * 2)(tab, row_p, col_p, es)
    return out_p[:e]


# speedup vs baseline: 5.6935x; 1.2213x over previous
import functools

import jax
import jax.numpy as jnp
from jax import lax
from jax.experimental import pallas as pl
from jax.experimental.pallas import tpu as pltpu
from jax.experimental.pallas import tpu_sc as plsc

N_WORKERS = 32
LANES = 16


def _node_scores_body(x_ref, w_ref, o_ref):
    o_ref[...] = jnp.dot(x_ref[...], w_ref[...],
                         preferred_element_type=jnp.float32)


def _edge_scores_body(e0_ref, e1_ref, e2_ref, e3_ref, w_ref, b_ref,
                      o0_ref, o1_ref, o2_ref, o3_ref):
    w = w_ref[...]
    bb = b_ref[...]
    o0_ref[...] = jnp.dot(e0_ref[0], w, preferred_element_type=jnp.float32) + bb
    o1_ref[...] = jnp.dot(e1_ref[0], w, preferred_element_type=jnp.float32) + bb
    o2_ref[...] = jnp.dot(e2_ref[0], w, preferred_element_type=jnp.float32) + bb
    o3_ref[...] = jnp.dot(e3_ref[0], w, preferred_element_type=jnp.float32) + bb


def _make_sc_gather(e_pad, n_tab):
    chunk = e_pad // N_WORKERS
    iters = chunk // LANES
    mesh = plsc.VectorSubcoreMesh(core_axis_name="c", subcore_axis_name="s")

    @functools.partial(
        pl.kernel,
        mesh=mesh,
        out_type=jax.ShapeDtypeStruct((e_pad,), jnp.float32),
        scratch_types=[
            pltpu.VMEM((n_tab,), jnp.float32),
            pltpu.VMEM((chunk,), jnp.int32),
            pltpu.VMEM((chunk,), jnp.int32),
            pltpu.VMEM((chunk,), jnp.float32),
            pltpu.VMEM((chunk,), jnp.float32),
        ],
        compiler_params=pltpu.CompilerParams(needs_layout_passes=False),
    )
    def sc_gather(tab_hbm, row_hbm, col_hbm, es_hbm, out_hbm,
                  tab_v, row_v, col_v, es_v, out_v):
        wid = lax.axis_index("s") * 2 + lax.axis_index("c")
        base = wid * chunk
        pltpu.sync_copy(tab_hbm, tab_v)
        pltpu.sync_copy(row_hbm.at[pl.ds(base, chunk)], row_v)
        pltpu.sync_copy(col_hbm.at[pl.ds(base, chunk)], col_v)
        pltpu.sync_copy(es_hbm.at[pl.ds(base, chunk)], es_v)

        def body(i, carry):
            off = i * LANES
            ir = row_v[pl.ds(off, LANES)]
            ic = col_v[pl.ds(off, LANES)]
            g_r = plsc.load_gather(tab_v, [ir * 2])
            g_c = plsc.load_gather(tab_v, [ic * 2 + 1])
            out_v[pl.ds(off, LANES)] = g_r + g_c + es_v[pl.ds(off, LANES)]
            return carry

        lax.fori_loop(0, iters, body, 0)
        pltpu.sync_copy(out_v, out_hbm.at[pl.ds(base, chunk)])

    return sc_gather


def kernel(x_embeddings, edge_embeddings, edge_index, W, b):
    n, h = x_embeddings.shape
    e = edge_embeddings.shape[0]

    row = edge_index[0].astype(jnp.int32)
    col = edge_index[1].astype(jnp.int32)
    w12 = jnp.concatenate([W[:h], W[h:2 * h]], axis=1)
    w3 = W[2 * h:]
    b2 = b.reshape(1, 1)

    n_blk = 2000
    ns2d = pl.pallas_call(
        _node_scores_body,
        grid=(n // n_blk,),
        in_specs=[
            pl.BlockSpec((n_blk, h), lambda i: (i, 0)),
            pl.BlockSpec((h, 2), lambda i: (0, 0)),
        ],
        out_specs=pl.BlockSpec((n_blk, 2), lambda i: (i, 0)),
        out_shape=jax.ShapeDtypeStruct((n, 2), jnp.float32),
    )(x_embeddings, w12)
    tab = ns2d.reshape(n * 2)

    e_pad = -(-e // 2560) * 2560
    n_str = 4
    q = e // n_str
    e_blk = 2000
    e4 = edge_embeddings.reshape(n_str, q, h)

    def _in_map(s):
        return lambda i, s=s: (s, i, 0)

    def _out_map():
        return lambda i: (i, 0)

    es_parts = pl.pallas_call(
        _edge_scores_body,
        grid=(q // e_blk,),
        in_specs=[pl.BlockSpec((1, e_blk, h), _in_map(s)) for s in range(n_str)]
        + [
            pl.BlockSpec((h, 1), lambda i: (0, 0)),
            pl.BlockSpec((1, 1), lambda i: (0, 0)),
        ],
        out_specs=[pl.BlockSpec((e_blk, 1), _out_map()) for _ in range(n_str)],
        out_shape=[jax.ShapeDtypeStruct((q, 1), jnp.float32)
                   for _ in range(n_str)],
    )(e4, e4, e4, e4, w3, b2)

    pad = e_pad - e
    es = jnp.concatenate([p.reshape(q) for p in es_parts]
                         + [jnp.zeros((pad,), jnp.float32)])

    zpad = jnp.zeros((pad,), jnp.int32)
    row_p = jnp.concatenate([row, zpad])
    col_p = jnp.concatenate([col, zpad])

    out_p = _make_sc_gather(e_pad, n * 2)(tab, row_p, col_p, es)
    return out_p[:e]
